# baseline (speedup 1.0000x reference)
import functools

import jax
import jax.numpy as jnp
from jax import lax
from jax.experimental import pallas as pl
from jax.experimental.pallas import tpu as pltpu
from jax.experimental.pallas import tpu_sc as plsc

B = 16384
V = 100000
D = 32
NCAT = 8
DDENSE = 13

NC = 2
NS = 16
NW = NC * NS
SLABW = 3200
LASTW = 768
TAILV = 99872
TAILC = 896
TSHIFT = TAILC + (V - D - TAILV) - LASTW
PIECE = 1024
NPIECE = B // PIECE

_mesh = plsc.VectorSubcoreMesh(
    core_axis_name="c", subcore_axis_name="s", num_cores=NC, num_subcores=NS
)


@functools.partial(
    pl.kernel,
    out_type=[jax.ShapeDtypeStruct((B, 128), jnp.float32) for _ in range(NCAT)],
    mesh=_mesh,
    scratch_types=[
        pltpu.VMEM((D, SLABW), jnp.float32),
        pltpu.VMEM((2, PIECE), jnp.int32),
        pltpu.VMEM((B + 80,), jnp.int32),
        pltpu.VMEM((4, 16, 128), jnp.float32),
        pltpu.VMEM((4, 16), jnp.int32),
        pltpu.SemaphoreType.DMA,
        pltpu.SemaphoreType.DMA,
        pltpu.SemaphoreType.DMA,
        pltpu.SemaphoreType.DMA,
        pltpu.SemaphoreType.DMA,
        pltpu.SemaphoreType.DMA,
    ],
    compiler_params=pltpu.CompilerParams(
        use_tc_tiling_on_sc=True, needs_layout_passes=False
    ),
)
def _gather8(cat0, cat1, cat2, cat3, cat4, cat5, cat6, cat7,
             et0, et1, et2, et3, et4, et5, et6, et7,
             tl0, tl1, tl2, tl3, tl4, tl5, tl6, tl7,
             o0, o1, o2, o3, o4, o5, o6, o7,
             slab_v, cat_v, list_v, row_v, pos_v,
             sem_slab, sem_cat, sem_s0, sem_s1, sem_s2, sem_s3):
    cats = [cat0, cat1, cat2, cat3, cat4, cat5, cat6, cat7]
    embts = [et0, et1, et2, et3, et4, et5, et6, et7]
    tails = [tl0, tl1, tl2, tl3, tl4, tl5, tl6, tl7]
    outs = [o0, o1, o2, o3, o4, o5, o6, o7]

    wid = lax.axis_index("s") * NC + lax.axis_index("c")
    c0 = wid * SLABW
    hi = jnp.minimum(c0 + SLABW, V)
    is_last = wid == NW - 1
    not_last = jnp.logical_not(is_last)
    lanes = lax.iota(jnp.int32, 16)

    def stage_slab(t):
        @pl.when(not_last)
        def _():
            pltpu.async_copy(
                embts[t].at[:, pl.ds(c0, SLABW)], slab_v, sem_slab
            )

        @pl.when(is_last)
        def _():
            pltpu.async_copy(
                embts[t].at[:, pl.ds(c0, LASTW)],
                slab_v.at[:, pl.ds(0, LASTW)],
                sem_slab,
            )
            pltpu.async_copy(
                tails[t], slab_v.at[:, pl.ds(TAILC, 128)], sem_slab
            )

    def wait_slab():
        @pl.when(not_last)
        def _():
            pltpu.make_async_copy(
                embts[0].at[:, pl.ds(0, SLABW)], slab_v, sem_slab
            ).wait()

        @pl.when(is_last)
        def _():
            pltpu.make_async_copy(
                embts[0].at[:, pl.ds(0, LASTW)],
                slab_v.at[:, pl.ds(0, LASTW)],
                sem_slab,
            ).wait()
            pltpu.make_async_copy(
                tails[0], slab_v.at[:, pl.ds(TAILC, 128)], sem_slab
            ).wait()

    def drain_scatter(t, sem):
        pltpu.make_async_copy(
            outs[t].at[pl.ds(0, 16)], row_v.at[0], sem
        ).wait()

    stage_slab(0)

    for t in range(NCAT):
        pltpu.async_copy(cats[t].at[pl.ds(0, PIECE)], cat_v.at[0], sem_cat)

        def piece_body(p, n_carry):
            pb = p % 2
            pltpu.make_async_copy(
                cats[t].at[pl.ds(0, PIECE)], cat_v.at[pb], sem_cat
            ).wait()

            @pl.when(p + 1 < NPIECE)
            def _():
                pltpu.async_copy(
                    cats[t].at[pl.ds((p + 1) * PIECE, PIECE)],
                    cat_v.at[(p + 1) % 2],
                    sem_cat,
                )

            width = (hi - c0).astype(jnp.uint32)

            def scan_body(cc, nn):
                for u in range(4):
                    iv = cat_v[pb, pl.ds(cc * 64 + u * 16, 16)]
                    li = iv - c0
                    m = li.astype(jnp.uint32) < width
                    bpos = p * PIECE + cc * 64 + u * 16 + lanes
                    packed = jnp.bitwise_or(jnp.left_shift(bpos, 12), li)
                    plsc.store_compressed(
                        list_v.at[pl.ds(nn, 16)], packed, mask=m
                    )
                    cnt = plsc.all_reduce_population_count(m)
                    cnt = cnt if cnt.ndim == 0 else cnt[0]
                    nn = nn + cnt
                return nn

            return lax.fori_loop(0, PIECE // 64, scan_body, n_carry)

        n = lax.fori_loop(0, NPIECE, piece_body, jnp.int32(0))

        wait_slab()

        @pl.when(n > 0)
        def _():
            e0v = list_v[pl.ds(0, 16)]
            pad = jnp.full((16,), e0v[0], jnp.int32)
            list_v[pl.ds(n, 16)] = pad
            list_v[pl.ds(n + 16, 16)] = pad
            list_v[pl.ds(n + 32, 16)] = pad
            list_v[pl.ds(n + 48, 16)] = pad
            nquads = (n + 63) // 64

            def do_chunk(c, slot, sem):
                packed = list_v[pl.ds(c * 16, 16)]
                bpos = jnp.right_shift(packed, 12)
                li = jnp.bitwise_and(packed, 4095)
                pos_v[slot] = bpos
                shift = jnp.logical_and(is_last, li >= LASTW)
                col = jnp.where(shift, li + TSHIFT, li)

                for jj in range(D):
                    jv = jnp.full((16,), jj, jnp.int32)
                    vals = plsc.load_gather(slab_v, [jv, col])
                    plsc.store_scatter(row_v.at[slot], [lanes, jv], vals)
                pltpu.async_copy(
                    row_v.at[slot], outs[t].at[pos_v.at[slot]], sem
                )

            sems = [sem_s0, sem_s1, sem_s2, sem_s3]

            def quad_body(cq, carry):
                for s in range(4):
                    @pl.when(cq >= 1)
                    def _():
                        drain_scatter(t, sems[s])

                    do_chunk(4 * cq + s, s, sems[s])
                return carry

            lax.fori_loop(0, nquads, quad_body, jnp.int32(0))
            for s in range(4):
                drain_scatter(t, sems[s])

        if t + 1 < NCAT:
            stage_slab(t + 1)


def kernel(cat0, cat1, cat2, cat3, cat4, cat5, cat6, cat7, dense,
           emb0, emb1, emb2, emb3, emb4, emb5, emb6, emb7):
    embs = [emb0, emb1, emb2, emb3, emb4, emb5, emb6, emb7]
    embts = [jnp.transpose(e) for e in embs]
    tails = [jnp.transpose(e[TAILV:]) for e in embs]
    gathered = _gather8(cat0, cat1, cat2, cat3, cat4, cat5, cat6, cat7,
                        *embts, *tails)
    return jnp.concatenate([*[g[:, :D] for g in gathered], dense], axis=-1)


# speedup vs baseline: 1.0514x; 1.0514x over previous
import functools

import jax
import jax.numpy as jnp
from jax import lax
from jax.experimental import pallas as pl
from jax.experimental.pallas import tpu as pltpu
from jax.experimental.pallas import tpu_sc as plsc

B = 16384
V = 100000
D = 32
NCAT = 8
DDENSE = 13

NC = 2
NS = 16
NW = NC * NS
SLABW = 3200
LASTW = 768
TAILV = 99872
TAILC = 896
TSHIFT = TAILC + (V - D - TAILV) - LASTW
PIECE = 2048
NPIECE = B // PIECE

_mesh = plsc.VectorSubcoreMesh(
    core_axis_name="c", subcore_axis_name="s", num_cores=NC, num_subcores=NS
)


@functools.partial(
    pl.kernel,
    out_type=[jax.ShapeDtypeStruct((B, 128), jnp.float32) for _ in range(NCAT)],
    mesh=_mesh,
    scratch_types=[
        pltpu.VMEM((D, SLABW), jnp.float32),
        pltpu.VMEM((2, PIECE), jnp.int32),
        pltpu.VMEM((B + 32,), jnp.int32),
        pltpu.VMEM((2, 16, 128), jnp.float32),
        pltpu.VMEM((2, 16), jnp.int32),
        pltpu.SemaphoreType.DMA,
        pltpu.SemaphoreType.DMA,
        pltpu.SemaphoreType.DMA,
        pltpu.SemaphoreType.DMA,
    ],
    compiler_params=pltpu.CompilerParams(
        use_tc_tiling_on_sc=True, needs_layout_passes=False
    ),
)
def _gather8(cat0, cat1, cat2, cat3, cat4, cat5, cat6, cat7,
             et0, et1, et2, et3, et4, et5, et6, et7,
             tl0, tl1, tl2, tl3, tl4, tl5, tl6, tl7,
             o0, o1, o2, o3, o4, o5, o6, o7,
             slab_v, cat_v, list_v, row_v, pos_v,
             sem_slab, sem_cat, sem_s0, sem_s1):
    cats = [cat0, cat1, cat2, cat3, cat4, cat5, cat6, cat7]
    embts = [et0, et1, et2, et3, et4, et5, et6, et7]
    tails = [tl0, tl1, tl2, tl3, tl4, tl5, tl6, tl7]
    outs = [o0, o1, o2, o3, o4, o5, o6, o7]

    wid = lax.axis_index("s") * NC + lax.axis_index("c")
    c0 = wid * SLABW
    hi = jnp.minimum(c0 + SLABW, V)
    is_last = wid == NW - 1
    not_last = jnp.logical_not(is_last)
    lanes = lax.iota(jnp.int32, 16)

    def stage_slab(t):
        @pl.when(not_last)
        def _():
            pltpu.async_copy(
                embts[t].at[:, pl.ds(c0, SLABW)], slab_v, sem_slab
            )

        @pl.when(is_last)
        def _():
            pltpu.async_copy(
                embts[t].at[:, pl.ds(c0, LASTW)],
                slab_v.at[:, pl.ds(0, LASTW)],
                sem_slab,
            )
            pltpu.async_copy(
                tails[t], slab_v.at[:, pl.ds(TAILC, 128)], sem_slab
            )

    def wait_slab():
        @pl.when(not_last)
        def _():
            pltpu.make_async_copy(
                embts[0].at[:, pl.ds(0, SLABW)], slab_v, sem_slab
            ).wait()

        @pl.when(is_last)
        def _():
            pltpu.make_async_copy(
                embts[0].at[:, pl.ds(0, LASTW)],
                slab_v.at[:, pl.ds(0, LASTW)],
                sem_slab,
            ).wait()
            pltpu.make_async_copy(
                tails[0], slab_v.at[:, pl.ds(TAILC, 128)], sem_slab
            ).wait()

    def drain_scatter(t, sem):
        pltpu.make_async_copy(
            outs[t].at[pl.ds(0, 16)], row_v.at[0], sem
        ).wait()

    stage_slab(0)

    for t in range(NCAT):
        pltpu.async_copy(cats[t].at[pl.ds(0, PIECE)], cat_v.at[0], sem_cat)

        def piece_body(p, n_carry):
            pb = p % 2
            pltpu.make_async_copy(
                cats[t].at[pl.ds(0, PIECE)], cat_v.at[pb], sem_cat
            ).wait()

            @pl.when(p + 1 < NPIECE)
            def _():
                pltpu.async_copy(
                    cats[t].at[pl.ds((p + 1) * PIECE, PIECE)],
                    cat_v.at[(p + 1) % 2],
                    sem_cat,
                )

            width = (hi - c0).astype(jnp.uint32)

            def scan_body(cc, nn):
                for u in range(4):
                    iv = cat_v[pb, pl.ds(cc * 64 + u * 16, 16)]
                    li = iv - c0
                    m = li.astype(jnp.uint32) < width
                    bpos = p * PIECE + cc * 64 + u * 16 + lanes
                    packed = jnp.bitwise_or(jnp.left_shift(bpos, 12), li)
                    plsc.store_compressed(
                        list_v.at[pl.ds(nn, 16)], packed, mask=m
                    )
                    cnt = plsc.all_reduce_population_count(m)
                    cnt = cnt if cnt.ndim == 0 else cnt[0]
                    nn = nn + cnt
                return nn

            return lax.fori_loop(0, PIECE // 64, scan_body, n_carry)

        n = lax.fori_loop(0, NPIECE, piece_body, jnp.int32(0))

        wait_slab()

        @pl.when(n > 0)
        def _():
            e0v = list_v[pl.ds(0, 16)]
            pad = jnp.full((16,), e0v[0], jnp.int32)
            list_v[pl.ds(n, 16)] = pad
            list_v[pl.ds(n + 16, 16)] = pad
            npairs = (n + 31) // 32

            def do_chunk(c, slot, sem):
                packed = list_v[pl.ds(c * 16, 16)]
                bpos = jnp.right_shift(packed, 12)
                li = jnp.bitwise_and(packed, 4095)
                pos_v[slot] = bpos
                shift = jnp.logical_and(is_last, li >= LASTW)
                col = jnp.where(shift, li + TSHIFT, li)

                for jj in range(D):
                    jv = jnp.full((16,), jj, jnp.int32)
                    vals = plsc.load_gather(slab_v, [jv, col])
                    plsc.store_scatter(row_v.at[slot], [lanes, jv], vals)
                pltpu.async_copy(
                    row_v.at[slot], outs[t].at[pos_v.at[slot]], sem
                )

            def pair_body(cp, carry):
                @pl.when(cp >= 1)
                def _():
                    drain_scatter(t, sem_s0)

                do_chunk(2 * cp, 0, sem_s0)

                @pl.when(cp >= 1)
                def _():
                    drain_scatter(t, sem_s1)

                do_chunk(2 * cp + 1, 1, sem_s1)
                return carry

            lax.fori_loop(0, npairs, pair_body, jnp.int32(0))
            drain_scatter(t, sem_s0)
            drain_scatter(t, sem_s1)

        if t + 1 < NCAT:
            stage_slab(t + 1)


def kernel(cat0, cat1, cat2, cat3, cat4, cat5, cat6, cat7, dense,
           emb0, emb1, emb2, emb3, emb4, emb5, emb6, emb7):
    embs = [emb0, emb1, emb2, emb3, emb4, emb5, emb6, emb7]
    embts = [jnp.transpose(e) for e in embs]
    tails = [jnp.transpose(e[TAILV:]) for e in embs]
    gathered = _gather8(cat0, cat1, cat2, cat3, cat4, cat5, cat6, cat7,
                        *embts, *tails)
    return jnp.concatenate([*[g[:, :D] for g in gathered], dense], axis=-1)
